# Initial kernel scaffold; baseline (speedup 1.0000x reference)
import jax
import jax.numpy as jnp
from jax.experimental import pallas as pl


def kernel(x, edge_index, W1, a_src1, a_dst1, b1, g1, be1, W2, a_src2, a_dst2, b2, g2, be2, W3, a_src3, a_dst3, b3):
    raise NotImplementedError("write your pallas kernel here")


# speedup vs baseline: 1.0135x; 1.0135x over previous
import functools

import jax
import jax.numpy as jnp
import numpy as np
from jax.experimental import pallas as pl

N = 10000
E = 320000
D_IN = 128
HID = 64
HEADS = 8
OUT = 128


def _matmul_kernel(x_ref, w_ref, o_ref):
    o_ref[...] = jnp.dot(x_ref[...], w_ref[...],
                         preferred_element_type=jnp.float32)


def _pallas_matmul(x, w, blk_m=1024):
    m, k = x.shape
    n = w.shape[1]
    pad_m = (-m) % blk_m
    if pad_m:
        x = jnp.pad(x, ((0, pad_m), (0, 0)))
    mp = x.shape[0]
    out = pl.pallas_call(
        _matmul_kernel,
        grid=(mp // blk_m,),
        in_specs=[
            pl.BlockSpec((blk_m, k), lambda i: (i, 0)),
            pl.BlockSpec((k, n), lambda i: (0, 0)),
        ],
        out_specs=pl.BlockSpec((blk_m, n), lambda i: (i, 0)),
        out_shape=jax.ShapeDtypeStruct((mp, n), jnp.float32),
    )(x, w)
    return out[:m]


def _gat_layer(x, src, dst, W, a_src, a_dst, bias, heads, out_ch, concat):
    n = x.shape[0]
    h = _pallas_matmul(x, W).reshape(n, heads, out_ch)
    alpha_src = (h * a_src[None, :, :]).sum(-1)
    alpha_dst = (h * a_dst[None, :, :]).sum(-1)
    e = alpha_src[src] + alpha_dst[dst]
    e = jnp.where(e > 0, e, 0.2 * e)
    m = jax.ops.segment_max(e, dst, num_segments=n)
    m = jnp.where(jnp.isfinite(m), m, 0.0)
    ex = jnp.exp(e - m[dst])
    s = jax.ops.segment_sum(ex, dst, num_segments=n)
    alpha = ex / (s[dst] + 1e-16)
    msg = alpha[:, :, None] * h[src]
    out = jax.ops.segment_sum(msg, dst, num_segments=n)
    if concat:
        out = out.reshape(n, heads * out_ch)
    else:
        out = out.mean(axis=1)
    return out + bias


def _batchnorm(x, gamma, beta):
    mean = x.mean(axis=0)
    var = x.var(axis=0)
    return gamma * (x - mean) / jnp.sqrt(var + 1e-5) + beta


def kernel(x, edge_index, W1, a_src1, a_dst1, b1, g1, be1,
           W2, a_src2, a_dst2, b2, g2, be2, W3, a_src3, a_dst3, b3):
    n = x.shape[0]
    loop = jnp.arange(n, dtype=edge_index.dtype)
    src = jnp.concatenate([edge_index[0], loop])
    dst = jnp.concatenate([edge_index[1], loop])
    h = _gat_layer(x, src, dst, W1, a_src1, a_dst1, b1, HEADS, HID, True)
    h = jax.nn.elu(_batchnorm(h, g1, be1))
    h = _gat_layer(h, src, dst, W2, a_src2, a_dst2, b2, HEADS, HID, True)
    h = jax.nn.elu(_batchnorm(h, g2, be2))
    h = _gat_layer(h, src, dst, W3, a_src3, a_dst3, b3, 1, OUT, False)
    return h


# speedup vs baseline: 1.8022x; 1.7782x over previous
import functools

import jax
import jax.numpy as jnp
import numpy as np
from jax import lax
from jax.experimental import pallas as pl
from jax.experimental.pallas import tpu as pltpu
from jax.experimental.pallas import tpu_sc as plsc

N = 10000
E = 320000
D_IN = 128
HID = 64
HEADS = 8
OUT = 128
F = HEADS * HID

NC, NS, L = 2, 16, 16
NW = NC * NS

EP = 330240
NPD = 10016
ACC_R = 10240
STRIPE = ACC_R // NS

BM = 1000
NBLK = N // BM

BSL = EP // 8
BB = 4128

CSL = EP // 16
CB = 240
CNB = CSL // CB

_f32 = jnp.float32
_i32 = jnp.int32


def _lrelu(x):
    return jnp.where(x > 0, x, 0.2 * x)


def _k1_body(x_ref, w_ref, pas_ref, pad_ref,
             hc0, hc1, hc2, hc3, as_ref, ad_ref, ms_ref, md_ref, c_ref):
    i = pl.program_id(0)
    h = jnp.dot(x_ref[...], w_ref[...], preferred_element_type=_f32)
    for k, hc in enumerate((hc0, hc1, hc2, hc3)):
        hc[...] = h[:, k * 128:(k + 1) * 128]
    dn = (((1,), (1,)), ((), ()))
    asb = lax.dot_general(h, pas_ref[...], dn, preferred_element_type=_f32,
                          precision=lax.Precision.HIGHEST)
    adb = lax.dot_general(h, pad_ref[...], dn, preferred_element_type=_f32,
                          precision=lax.Precision.HIGHEST)
    as_ref[...] = asb
    ad_ref[...] = adb

    @pl.when(i == 0)
    def _():
        ms_ref[...] = jnp.full((HEADS, HEADS), -1e30, _f32)
        md_ref[...] = jnp.full((HEADS, HEADS), -1e30, _f32)

    ms_ref[...] = jnp.maximum(
        ms_ref[...], jnp.max(asb, axis=0, keepdims=True))
    md_ref[...] = jnp.maximum(
        md_ref[...], jnp.max(adb, axis=0, keepdims=True))

    @pl.when(i == NBLK - 1)
    def _():
        c_ref[...] = _lrelu(ms_ref[...] + md_ref[...])


def _k1(x, w, pas, pad):
    kdim = x.shape[1]
    outs = (
        [jax.ShapeDtypeStruct((N, 128), _f32) for _ in range(4)]
        + [jax.ShapeDtypeStruct((N, HEADS), _f32) for _ in range(2)]
        + [jax.ShapeDtypeStruct((HEADS, HEADS), _f32) for _ in range(3)]
    )
    res = pl.pallas_call(
        _k1_body,
        grid=(NBLK,),
        in_specs=[
            pl.BlockSpec((BM, kdim), lambda i: (i, 0)),
            pl.BlockSpec((kdim, F), lambda i: (0, 0)),
            pl.BlockSpec((HEADS, F), lambda i: (0, 0)),
            pl.BlockSpec((HEADS, F), lambda i: (0, 0)),
        ],
        out_specs=(
            [pl.BlockSpec((BM, 128), lambda i: (i, 0)) for _ in range(4)]
            + [pl.BlockSpec((BM, HEADS), lambda i: (i, 0)) for _ in range(2)]
            + [pl.BlockSpec((HEADS, HEADS), lambda i: (0, 0)) for _ in range(3)]
        ),
        out_shape=outs,
    )(x, w, pas, pad)
    return res[0:4], res[4], res[5], res[8]


def _phase_b_body(asT, adT, cflat, src_hbm, dst_hbm,
                  ex0, ex1, ex2, ex3, s_hbm,
                  tas0, tas1, tad0, tad1, cbuf, srcb, dstb,
                  exb0, exb1, sP0, sP1):
    core = lax.axis_index("c")
    sub = lax.axis_index("s")
    t = core * NS + sub
    chunk = t // 8
    sl = t % 8

    def _z(k, _):
        z = jnp.zeros((L,), _f32)
        zs = pl.ds(k * L, L)
        sP0[zs] = z
        sP1[zs] = z
        return 0
    lax.fori_loop(0, NPD // L, _z, 0)

    for ck, ex_out in enumerate((ex0, ex1, ex2, ex3)):
        @pl.when(chunk == ck)
        def _(ck=ck, ex_out=ex_out):
            pltpu.sync_copy(asT.at[ck, 0], tas0)
            pltpu.sync_copy(asT.at[ck, 1], tas1)
            pltpu.sync_copy(adT.at[ck, 0], tad0)
            pltpu.sync_copy(adT.at[ck, 1], tad1)
            pltpu.sync_copy(cflat.at[pl.ds(0, L)], cbuf)
            cv = cbuf[pl.ds(0, L)]
            c0 = jnp.broadcast_to(cv[2 * ck], (L,))
            c1 = jnp.broadcast_to(cv[2 * ck + 1], (L,))
            base0 = sl * BSL

            def _batch(bi, _):
                eb = pl.multiple_of(base0 + bi * BB, 8)
                pltpu.sync_copy(src_hbm.at[pl.ds(eb, BB)], srcb)
                pltpu.sync_copy(dst_hbm.at[pl.ds(eb, BB)], dstb)

                def _vec(k, _):
                    ks = pl.ds(k * L, L)
                    srcv = srcb[ks]
                    dstv = dstb[ks]
                    for tas, tad, cc, exb, sP in (
                            (tas0, tad0, c0, exb0, sP0),
                            (tas1, tad1, c1, exb1, sP1)):
                        va = plsc.load_gather(tas, [srcv])
                        vd = plsc.load_gather(tad, [dstv])
                        exv = jnp.exp(_lrelu(va + vd) - cc)
                        exb[ks] = exv
                        plsc.addupdate_scatter(sP, [dstv], exv)
                    return 0
                lax.fori_loop(0, BB // L, _vec, 0)
                for hl, exb in ((0, exb0), (1, exb1)):
                    xo = pl.multiple_of((sl * 2 + hl) * BSL + bi * BB, 8)
                    pltpu.sync_copy(exb, ex_out.at[pl.ds(xo, BB)])
                return 0
            lax.fori_loop(0, BSL // BB, _batch, 0)

    pltpu.sync_copy(sP0, s_hbm.at[t, 0])
    pltpu.sync_copy(sP1, s_hbm.at[t, 1])


def _phase_b(asT, adT, cflat, src_i, dst_i):
    mesh = plsc.VectorSubcoreMesh(
        core_axis_name="c", subcore_axis_name="s",
        num_cores=NC, num_subcores=NS)
    outs = (
        [jax.ShapeDtypeStruct((2 * EP,), _f32) for _ in range(4)]
        + [jax.ShapeDtypeStruct((NW, 2, NPD), _f32)]
    )
    fn = pl.kernel(
        _phase_b_body,
        out_type=outs,
        mesh=mesh,
        compiler_params=pltpu.CompilerParams(needs_layout_passes=False),
        scratch_types=[
            pltpu.VMEM((N,), _f32),
            pltpu.VMEM((N,), _f32),
            pltpu.VMEM((N,), _f32),
            pltpu.VMEM((N,), _f32),
            pltpu.VMEM((L,), _f32),
            pltpu.VMEM((BB,), _i32),
            pltpu.VMEM((BB,), _i32),
            pltpu.VMEM((BB,), _f32),
            pltpu.VMEM((BB,), _f32),
            pltpu.VMEM((NPD,), _f32),
            pltpu.VMEM((NPD,), _f32),
        ],
    )
    return fn(asT, adT, cflat, src_i, dst_i)


def _phase_c_body(hc0, hc1, hc2, hc3, ex0, ex1, ex2, ex3,
                  src_hbm, dst_hbm, zeros_hbm, out_hbm,
                  acc, rows, srcb, dstb, exb0, exb1, sem):
    core = lax.axis_index("c")
    sub = lax.axis_index("s")
    stripe = pl.ds(pl.multiple_of(sub * STRIPE, STRIPE), STRIPE)
    hcs = (hc0, hc1, hc2, hc3)
    exs = (ex0, ex1, ex2, ex3)

    for p in range(2):
        pltpu.sync_copy(zeros_hbm.at[stripe], acc.at[stripe])
        plsc.subcore_barrier()

        for ck in range(4):
            if ck % 2 != p:
                continue

            @pl.when(core == ck // 2)
            def _(ck=ck):
                hc = hcs[ck]
                ex = exs[ck]
                base0 = sub * CSL
                bsl = sub // 2
                boff = (sub % 2) * CSL

                def _batch(bi, _):
                    eb = pl.multiple_of(base0 + bi * CB, 8)
                    pltpu.sync_copy(src_hbm.at[pl.ds(eb, CB)], srcb)
                    pltpu.sync_copy(dst_hbm.at[pl.ds(eb, CB)], dstb)
                    off = boff + bi * CB
                    for hl, exb in ((0, exb0), (1, exb1)):
                        xo = pl.multiple_of((bsl * 2 + hl) * BSL + off, 8)
                        pltpu.sync_copy(ex.at[pl.ds(xo, CB)], exb)
                    pltpu.async_copy(hc.at[srcb], rows, sem).wait()

                    def _edge16(k, _):
                        ks = pl.ds(k * L, L)
                        ex0v = exb0[ks]
                        ex1v = exb1[ks]
                        for i in range(L):
                            e = k * L + i
                            x0 = jnp.broadcast_to(ex0v[i], (L,))
                            x1 = jnp.broadcast_to(ex1v[i], (L,))
                            for j in range(8):
                                js = pl.ds(j * L, L)
                                v = rows[e, js]
                                rows[e, js] = v * (x0 if j < 4 else x1)
                        return 0
                    lax.fori_loop(0, CB // L, _edge16, 0)
                    pltpu.sync_copy(rows, acc.at[dstb], add=True)
                    return 0
                lax.fori_loop(0, CNB, _batch, 0)

        plsc.subcore_barrier()
        col = pl.multiple_of((2 * core + p) * 128, 128)
        pltpu.sync_copy(
            acc.at[stripe], out_hbm.at[stripe, pl.ds(col, 128)])
        plsc.subcore_barrier()


def _phase_c(hcs, exs, src_i, dst_i, zeros_acc):
    mesh = plsc.VectorSubcoreMesh(
        core_axis_name="c", subcore_axis_name="s",
        num_cores=NC, num_subcores=NS)
    fn = pl.kernel(
        _phase_c_body,
        out_type=jax.ShapeDtypeStruct((ACC_R, F), _f32),
        mesh=mesh,
        compiler_params=pltpu.CompilerParams(needs_layout_passes=False),
        scratch_types=[
            pltpu.VMEM_SHARED((ACC_R, 128), _f32),
            pltpu.VMEM((CB, 128), _f32),
            pltpu.VMEM((CB,), _i32),
            pltpu.VMEM((CB,), _i32),
            pltpu.VMEM((CB,), _f32),
            pltpu.VMEM((CB,), _f32),
            pltpu.SemaphoreType.DMA,
        ],
    )
    return fn(*hcs, *exs, src_i, dst_i, zeros_acc)


def _proj_mats(a_src, a_dst):
    eye = jnp.eye(HEADS, dtype=_f32)
    pas = (eye[:, :, None] * a_src[None, :, :]).reshape(HEADS, F)
    pad = (eye[:, :, None] * a_dst[None, :, :]).reshape(HEADS, F)
    return pas, pad


def _edge_arrays(edge_index):
    src = edge_index[0].astype(_i32)
    dst = edge_index[1].astype(_i32)
    loop = jnp.arange(N, dtype=_i32)
    pad = EP - (E + N)
    src_i = jnp.concatenate([src, loop, jnp.zeros((pad,), _i32)])
    dst_i = jnp.concatenate([dst, loop, jnp.full((pad,), N, _i32)])
    return src_i, dst_i


def _combine_s(s_hbm):
    s4 = s_hbm.reshape(4, 8, 2, NPD).sum(axis=1)
    return s4.reshape(8, NPD)[:, :N].T


def kernel(x, edge_index, W1, a_src1, a_dst1, b1, g1, be1,
           W2, a_src2, a_dst2, b2, g2, be2, W3, a_src3, a_dst3, b3):
    src_i, dst_i = _edge_arrays(edge_index)
    zeros_acc = jnp.zeros((ACC_R, 128), _f32)

    pas1, pad1 = _proj_mats(a_src1, a_dst1)
    hcs1, as1, ad1, c1 = _k1(x, W1, pas1, pad1)
    asT1 = as1.T.reshape(4, 2, N)
    adT1 = ad1.T.reshape(4, 2, N)
    cflat1 = c1.reshape(64)
    *exs1, s1 = _phase_b(asT1, adT1, cflat1, src_i, dst_i)
    acc1 = _phase_c(hcs1, exs1, src_i, dst_i, zeros_acc)

    s1c = _combine_s(s1)
    rinv1 = 1.0 / (s1c + 1e-16)
    out1 = acc1[:N] * jnp.repeat(rinv1, HID, axis=1) + b1

    loop = jnp.arange(N, dtype=_i32)
    src = jnp.concatenate([edge_index[0].astype(_i32), loop])
    dst = jnp.concatenate([edge_index[1].astype(_i32), loop])

    h = jax.nn.elu(_batchnorm(out1, g1, be1))
    h = _gat_layer_xla(h, src, dst, W2, a_src2, a_dst2, b2, HEADS, HID, True)
    h = jax.nn.elu(_batchnorm(h, g2, be2))
    h = _gat_layer_xla(h, src, dst, W3, a_src3, a_dst3, b3, 1, OUT, False)
    return h


def _gat_layer_xla(xin, src, dst, W, a_src, a_dst, bias, heads, out_ch, concat):
    n = xin.shape[0]
    h = (xin @ W).reshape(n, heads, out_ch)
    alpha_src = (h * a_src[None, :, :]).sum(-1)
    alpha_dst = (h * a_dst[None, :, :]).sum(-1)
    e = alpha_src[src] + alpha_dst[dst]
    e = _lrelu(e)
    m = jax.ops.segment_max(e, dst, num_segments=n)
    m = jnp.where(jnp.isfinite(m), m, 0.0)
    ex = jnp.exp(e - m[dst])
    s = jax.ops.segment_sum(ex, dst, num_segments=n)
    alpha = ex / (s[dst] + 1e-16)
    msg = alpha[:, :, None] * h[src]
    o = jax.ops.segment_sum(msg, dst, num_segments=n)
    if concat:
        o = o.reshape(n, heads * out_ch)
    else:
        o = o.mean(axis=1)
    return o + bias


def _batchnorm(xin, gamma, beta):
    mean = xin.mean(axis=0)
    var = xin.var(axis=0)
    return gamma * (xin - mean) / jnp.sqrt(var + 1e-5) + beta


# speedup vs baseline: 31.8115x; 17.6516x over previous
import functools

import jax
import jax.numpy as jnp
import numpy as np
from jax import lax
from jax.experimental import pallas as pl
from jax.experimental.pallas import tpu as pltpu
from jax.experimental.pallas import tpu_sc as plsc

N = 10000
E = 320000
D_IN = 128
HID = 64
HEADS = 8
OUT = 128
F = HEADS * HID

NC, NS, L = 2, 16, 16
NW = NC * NS

EP = 330240
NPD = 10016
ACC_R = 10240
STRIPE = ACC_R // NS

BM = 1000
NBLK = N // BM

BSL = EP // 8
BB = 4128

CSL = EP // 16
CB = 240
CNB = CSL // CB

_f32 = jnp.float32
_i32 = jnp.int32


def _lrelu(x):
    return jnp.where(x > 0, x, 0.2 * x)


def _k1_body(x_ref, w_ref, pas_ref, pad_ref,
             hc0, hc1, hc2, hc3, as_ref, ad_ref, ms_ref, md_ref, c_ref):
    i = pl.program_id(0)
    h = jnp.dot(x_ref[...], w_ref[...], preferred_element_type=_f32)
    for k, hc in enumerate((hc0, hc1, hc2, hc3)):
        hc[...] = h[:, k * 128:(k + 1) * 128]
    dn = (((1,), (1,)), ((), ()))
    asb = lax.dot_general(h, pas_ref[...], dn, preferred_element_type=_f32,
                          precision=lax.Precision.HIGHEST)
    adb = lax.dot_general(h, pad_ref[...], dn, preferred_element_type=_f32,
                          precision=lax.Precision.HIGHEST)
    as_ref[...] = asb
    ad_ref[...] = adb

    @pl.when(i == 0)
    def _():
        ms_ref[...] = jnp.full((HEADS, HEADS), -1e30, _f32)
        md_ref[...] = jnp.full((HEADS, HEADS), -1e30, _f32)

    ms_ref[...] = jnp.maximum(
        ms_ref[...], jnp.max(asb, axis=0, keepdims=True))
    md_ref[...] = jnp.maximum(
        md_ref[...], jnp.max(adb, axis=0, keepdims=True))

    @pl.when(i == NBLK - 1)
    def _():
        c_ref[...] = _lrelu(ms_ref[...] + md_ref[...])


def _k1(x, w, pas, pad):
    kdim = x.shape[1]
    outs = (
        [jax.ShapeDtypeStruct((N, 128), _f32) for _ in range(4)]
        + [jax.ShapeDtypeStruct((N, HEADS), _f32) for _ in range(2)]
        + [jax.ShapeDtypeStruct((HEADS, HEADS), _f32) for _ in range(3)]
    )
    res = pl.pallas_call(
        _k1_body,
        grid=(NBLK,),
        in_specs=[
            pl.BlockSpec((BM, kdim), lambda i: (i, 0)),
            pl.BlockSpec((kdim, F), lambda i: (0, 0)),
            pl.BlockSpec((HEADS, F), lambda i: (0, 0)),
            pl.BlockSpec((HEADS, F), lambda i: (0, 0)),
        ],
        out_specs=(
            [pl.BlockSpec((BM, 128), lambda i: (i, 0)) for _ in range(4)]
            + [pl.BlockSpec((BM, HEADS), lambda i: (i, 0)) for _ in range(2)]
            + [pl.BlockSpec((HEADS, HEADS), lambda i: (0, 0)) for _ in range(3)]
        ),
        out_shape=outs,
    )(x, w, pas, pad)
    return res[0:4], res[4], res[5], res[8]


def _phase_b_body(asT, adT, cflat, src_hbm, dst_hbm,
                  ex0, ex1, ex2, ex3, s_hbm,
                  tas0, tas1, tad0, tad1, cbuf, srcb, dstb,
                  exb0, exb1, sP0, sP1):
    core = lax.axis_index("c")
    sub = lax.axis_index("s")
    t = core * NS + sub
    chunk = t // 8
    sl = t % 8

    def _z(k, _):
        z = jnp.zeros((L,), _f32)
        zs = pl.ds(k * L, L)
        sP0[zs] = z
        sP1[zs] = z
        return 0
    lax.fori_loop(0, NPD // L, _z, 0)

    for ck, ex_out in enumerate((ex0, ex1, ex2, ex3)):
        @pl.when(chunk == ck)
        def _(ck=ck, ex_out=ex_out):
            pltpu.sync_copy(asT.at[ck, 0], tas0)
            pltpu.sync_copy(asT.at[ck, 1], tas1)
            pltpu.sync_copy(adT.at[ck, 0], tad0)
            pltpu.sync_copy(adT.at[ck, 1], tad1)
            pltpu.sync_copy(cflat.at[pl.ds(0, L)], cbuf)
            cv = cbuf[pl.ds(0, L)]
            c0 = jnp.broadcast_to(cv[2 * ck], (L,))
            c1 = jnp.broadcast_to(cv[2 * ck + 1], (L,))
            base0 = sl * BSL

            def _batch(bi, _):
                eb = pl.multiple_of(base0 + bi * BB, 8)
                pltpu.sync_copy(src_hbm.at[pl.ds(eb, BB)], srcb)
                pltpu.sync_copy(dst_hbm.at[pl.ds(eb, BB)], dstb)

                def _vec(k, _):
                    ks = pl.ds(k * L, L)
                    srcv = srcb[ks]
                    dstv = dstb[ks]
                    for tas, tad, cc, exb, sP in (
                            (tas0, tad0, c0, exb0, sP0),
                            (tas1, tad1, c1, exb1, sP1)):
                        va = plsc.load_gather(tas, [srcv])
                        vd = plsc.load_gather(tad, [dstv])
                        exv = jnp.exp(_lrelu(va + vd) - cc)
                        exb[ks] = exv
                        plsc.addupdate_scatter(sP, [dstv], exv)
                    return 0
                lax.fori_loop(0, BB // L, _vec, 0)
                for hl, exb in ((0, exb0), (1, exb1)):
                    xo = pl.multiple_of((sl * 2 + hl) * BSL + bi * BB, 8)
                    pltpu.sync_copy(exb, ex_out.at[pl.ds(xo, BB)])
                return 0
            lax.fori_loop(0, BSL // BB, _batch, 0)

    pltpu.sync_copy(sP0, s_hbm.at[t, 0])
    pltpu.sync_copy(sP1, s_hbm.at[t, 1])


def _phase_b(asT, adT, cflat, src_i, dst_i):
    mesh = plsc.VectorSubcoreMesh(
        core_axis_name="c", subcore_axis_name="s",
        num_cores=NC, num_subcores=NS)
    outs = (
        [jax.ShapeDtypeStruct((2 * EP,), _f32) for _ in range(4)]
        + [jax.ShapeDtypeStruct((NW, 2, NPD), _f32)]
    )
    fn = pl.kernel(
        _phase_b_body,
        out_type=outs,
        mesh=mesh,
        compiler_params=pltpu.CompilerParams(needs_layout_passes=False),
        scratch_types=[
            pltpu.VMEM((N,), _f32),
            pltpu.VMEM((N,), _f32),
            pltpu.VMEM((N,), _f32),
            pltpu.VMEM((N,), _f32),
            pltpu.VMEM((L,), _f32),
            pltpu.VMEM((BB,), _i32),
            pltpu.VMEM((BB,), _i32),
            pltpu.VMEM((BB,), _f32),
            pltpu.VMEM((BB,), _f32),
            pltpu.VMEM((NPD,), _f32),
            pltpu.VMEM((NPD,), _f32),
        ],
    )
    return fn(asT, adT, cflat, src_i, dst_i)


def _phase_c_body(hc0, hc1, hc2, hc3, ex0, ex1, ex2, ex3,
                  src_hbm, dst_hbm, zeros_hbm, out_hbm,
                  acc, rows, srcb, dstb, exb0, exb1, sem):
    core = lax.axis_index("c")
    sub = lax.axis_index("s")
    stripe = pl.ds(pl.multiple_of(sub * STRIPE, STRIPE), STRIPE)
    hcs = (hc0, hc1, hc2, hc3)
    exs = (ex0, ex1, ex2, ex3)

    for p in range(2):
        pltpu.sync_copy(zeros_hbm.at[stripe], acc.at[stripe])
        plsc.subcore_barrier()

        for ck in range(4):
            if ck % 2 != p:
                continue

            @pl.when(core == ck // 2)
            def _(ck=ck):
                hc = hcs[ck]
                ex = exs[ck]
                base0 = sub * CSL
                bsl = sub // 2
                boff = (sub % 2) * CSL

                def _batch(bi, _):
                    eb = pl.multiple_of(base0 + bi * CB, 8)
                    pltpu.sync_copy(src_hbm.at[pl.ds(eb, CB)], srcb)
                    pltpu.sync_copy(dst_hbm.at[pl.ds(eb, CB)], dstb)
                    off = boff + bi * CB
                    for hl, exb in ((0, exb0), (1, exb1)):
                        xo = pl.multiple_of((bsl * 2 + hl) * BSL + off, 8)
                        pltpu.sync_copy(ex.at[pl.ds(xo, CB)], exb)
                    pltpu.async_copy(hc.at[srcb], rows, sem).wait()

                    def _edge16(k, _):
                        ks = pl.ds(k * L, L)
                        ex0v = exb0[ks]
                        ex1v = exb1[ks]
                        for i in range(L):
                            e = k * L + i
                            x0 = jnp.broadcast_to(ex0v[i], (L,))
                            x1 = jnp.broadcast_to(ex1v[i], (L,))
                            for j in range(8):
                                js = pl.ds(j * L, L)
                                v = rows[e, js]
                                rows[e, js] = v * (x0 if j < 4 else x1)
                        return 0
                    lax.fori_loop(0, CB // L, _edge16, 0)
                    pltpu.sync_copy(rows, acc.at[dstb], add=True)
                    return 0
                lax.fori_loop(0, CNB, _batch, 0)

        plsc.subcore_barrier()
        col = pl.multiple_of((2 * core + p) * 128, 128)
        pltpu.sync_copy(
            acc.at[stripe], out_hbm.at[stripe, pl.ds(col, 128)])
        plsc.subcore_barrier()


def _phase_c(hcs, exs, src_i, dst_i, zeros_acc):
    mesh = plsc.VectorSubcoreMesh(
        core_axis_name="c", subcore_axis_name="s",
        num_cores=NC, num_subcores=NS)
    fn = pl.kernel(
        _phase_c_body,
        out_type=jax.ShapeDtypeStruct((ACC_R, F), _f32),
        mesh=mesh,
        compiler_params=pltpu.CompilerParams(needs_layout_passes=False),
        scratch_types=[
            pltpu.VMEM_SHARED((ACC_R, 128), _f32),
            pltpu.VMEM((CB, 128), _f32),
            pltpu.VMEM((CB,), _i32),
            pltpu.VMEM((CB,), _i32),
            pltpu.VMEM((CB,), _f32),
            pltpu.VMEM((CB,), _f32),
            pltpu.SemaphoreType.DMA,
        ],
    )
    return fn(*hcs, *exs, src_i, dst_i, zeros_acc)


def _k2_body(nout, acc_ref, s_ref, b_ref, g_ref, be_ref, w_ref,
             pas_ref, pad_ref, erep_ref, *out_refs):
    p = pl.program_id(0)
    i = pl.program_id(1)
    hc_refs = out_refs[:nout]
    as_ref, ad_ref, ms_ref, md_ref, c_ref, ssum_ref, ssq_ref = out_refs[nout:]

    rinv = 1.0 / (s_ref[...] + 1e-16)
    expand = jnp.dot(rinv, erep_ref[...], preferred_element_type=_f32,
                     precision=lax.Precision.HIGHEST)
    z = acc_ref[...] * expand + b_ref[...]

    @pl.when(jnp.logical_and(p == 0, i == 0))
    def _():
        ssum_ref[...] = jnp.zeros((HEADS, F), _f32)
        ssq_ref[...] = jnp.zeros((HEADS, F), _f32)

    @pl.when(p == 0)
    def _():
        ssum_ref[...] += jnp.broadcast_to(
            jnp.sum(z, axis=0, keepdims=True), (HEADS, F))
        ssq_ref[...] += jnp.broadcast_to(
            jnp.sum(z * z, axis=0, keepdims=True), (HEADS, F))

    @pl.when(p == 1)
    def _():
        mean = ssum_ref[...][0:1, :] * (1.0 / N)
        var = ssq_ref[...][0:1, :] * (1.0 / N) - mean * mean
        xn = g_ref[...] * (z - mean) * lax.rsqrt(var + 1e-5) + be_ref[...]
        y = jnp.where(xn > 0, xn, jnp.exp(xn) - 1.0)
        hn = jnp.dot(y, w_ref[...], preferred_element_type=_f32)
        fo = hn.shape[1]
        for k, hc in enumerate(hc_refs):
            hc[...] = hn[:, k * 128:(k + 1) * 128]
        dn = (((1,), (1,)), ((), ()))
        asb = lax.dot_general(hn, pas_ref[...], dn,
                              preferred_element_type=_f32,
                              precision=lax.Precision.HIGHEST)
        adb = lax.dot_general(hn, pad_ref[...], dn,
                              preferred_element_type=_f32,
                              precision=lax.Precision.HIGHEST)
        as_ref[...] = asb
        ad_ref[...] = adb

        @pl.when(i == 0)
        def _():
            ms_ref[...] = jnp.full((HEADS, HEADS), -1e30, _f32)
            md_ref[...] = jnp.full((HEADS, HEADS), -1e30, _f32)

        ms_ref[...] = jnp.maximum(
            ms_ref[...], jnp.max(asb, axis=0, keepdims=True))
        md_ref[...] = jnp.maximum(
            md_ref[...], jnp.max(adb, axis=0, keepdims=True))

        @pl.when(i == NBLK - 1)
        def _():
            c_ref[...] = _lrelu(ms_ref[...] + md_ref[...])


def _k2(acc, s_comb, b, g, be, w, pas, pad, erep, nout):
    fo = w.shape[1]
    outs = (
        [jax.ShapeDtypeStruct((N, 128), _f32) for _ in range(nout)]
        + [jax.ShapeDtypeStruct((N, HEADS), _f32) for _ in range(2)]
        + [jax.ShapeDtypeStruct((HEADS, HEADS), _f32) for _ in range(3)]
        + [jax.ShapeDtypeStruct((HEADS, F), _f32) for _ in range(2)]
    )
    res = pl.pallas_call(
        functools.partial(_k2_body, nout),
        grid=(2, NBLK),
        in_specs=[
            pl.BlockSpec((BM, F), lambda p, i: (i, 0)),
            pl.BlockSpec((BM, HEADS), lambda p, i: (i, 0)),
            pl.BlockSpec((1, F), lambda p, i: (0, 0)),
            pl.BlockSpec((1, F), lambda p, i: (0, 0)),
            pl.BlockSpec((1, F), lambda p, i: (0, 0)),
            pl.BlockSpec((F, fo), lambda p, i: (0, 0)),
            pl.BlockSpec((HEADS, fo), lambda p, i: (0, 0)),
            pl.BlockSpec((HEADS, fo), lambda p, i: (0, 0)),
            pl.BlockSpec((HEADS, F), lambda p, i: (0, 0)),
        ],
        out_specs=(
            [pl.BlockSpec((BM, 128), lambda p, i: (i, 0))
             for _ in range(nout)]
            + [pl.BlockSpec((BM, HEADS), lambda p, i: (i, 0))
               for _ in range(2)]
            + [pl.BlockSpec((HEADS, HEADS), lambda p, i: (0, 0))
               for _ in range(3)]
            + [pl.BlockSpec((HEADS, F), lambda p, i: (0, 0))
               for _ in range(2)]
        ),
        out_shape=outs,
    )(acc[:N], s_comb, b.reshape(1, F), g.reshape(1, F),
      be.reshape(1, F), w, pas, pad, erep)
    return res[:nout], res[nout], res[nout + 1], res[nout + 4]


SL3 = EP // NW


def _phase_b3_body(asf, adf, cflat, src_hbm, dst_hbm, ex_out, s_hbm,
                   tas0, tad0, cbuf, srcb, dstb, exb, sP0):
    core = lax.axis_index("c")
    sub = lax.axis_index("s")
    t = core * NS + sub

    def _z(k, _):
        sP0[pl.ds(k * L, L)] = jnp.zeros((L,), _f32)
        return 0
    lax.fori_loop(0, NPD // L, _z, 0)

    pltpu.sync_copy(asf, tas0)
    pltpu.sync_copy(adf, tad0)
    pltpu.sync_copy(cflat.at[pl.ds(0, L)], cbuf)
    cv = cbuf[pl.ds(0, L)]
    c0 = jnp.broadcast_to(cv[0], (L,))

    eb = pl.multiple_of(t * SL3, 8)
    pltpu.sync_copy(src_hbm.at[pl.ds(eb, SL3)], srcb)
    pltpu.sync_copy(dst_hbm.at[pl.ds(eb, SL3)], dstb)

    def _vec(k, _):
        ks = pl.ds(k * L, L)
        srcv = srcb[ks]
        dstv = dstb[ks]
        va = plsc.load_gather(tas0, [srcv])
        vd = plsc.load_gather(tad0, [dstv])
        exv = jnp.exp(_lrelu(va + vd) - c0)
        exb[ks] = exv
        plsc.addupdate_scatter(sP0, [dstv], exv)
        return 0
    lax.fori_loop(0, SL3 // L, _vec, 0)

    pltpu.sync_copy(exb, ex_out.at[pl.ds(eb, SL3)])
    pltpu.sync_copy(sP0, s_hbm.at[t])


def _phase_b3(asf, adf, cflat, src_i, dst_i):
    mesh = plsc.VectorSubcoreMesh(
        core_axis_name="c", subcore_axis_name="s",
        num_cores=NC, num_subcores=NS)
    outs = [jax.ShapeDtypeStruct((EP,), _f32),
            jax.ShapeDtypeStruct((NW, NPD), _f32)]
    fn = pl.kernel(
        _phase_b3_body,
        out_type=outs,
        mesh=mesh,
        compiler_params=pltpu.CompilerParams(needs_layout_passes=False),
        scratch_types=[
            pltpu.VMEM((N,), _f32),
            pltpu.VMEM((N,), _f32),
            pltpu.VMEM((L,), _f32),
            pltpu.VMEM((SL3,), _i32),
            pltpu.VMEM((SL3,), _i32),
            pltpu.VMEM((SL3,), _f32),
            pltpu.VMEM((NPD,), _f32),
        ],
    )
    return fn(asf, adf, cflat, src_i, dst_i)


CNB3 = SL3 // CB


def _phase_c3_body(hc, ex, src_hbm, dst_hbm, zeros_hbm, out_hbm,
                   acc, rows, srcb, dstb, exb0, sem):
    core = lax.axis_index("c")
    sub = lax.axis_index("s")
    t = core * NS + sub
    stripe = pl.ds(pl.multiple_of(sub * STRIPE, STRIPE), STRIPE)

    pltpu.sync_copy(zeros_hbm.at[stripe], acc.at[stripe])
    plsc.subcore_barrier()

    base0 = t * SL3

    def _batch(bi, _):
        eb = pl.multiple_of(base0 + bi * CB, 8)
        pltpu.sync_copy(src_hbm.at[pl.ds(eb, CB)], srcb)
        pltpu.sync_copy(dst_hbm.at[pl.ds(eb, CB)], dstb)
        pltpu.sync_copy(ex.at[pl.ds(eb, CB)], exb0)
        pltpu.async_copy(hc.at[srcb], rows, sem).wait()

        def _edge16(k, _):
            ks = pl.ds(k * L, L)
            ex0v = exb0[ks]
            for i in range(L):
                e = k * L + i
                x0 = jnp.broadcast_to(ex0v[i], (L,))
                for j in range(8):
                    js = pl.ds(j * L, L)
                    v = rows[e, js]
                    rows[e, js] = v * x0
            return 0
        lax.fori_loop(0, CB // L, _edge16, 0)
        pltpu.sync_copy(rows, acc.at[dstb], add=True)
        return 0
    lax.fori_loop(0, CNB3, _batch, 0)

    plsc.subcore_barrier()
    pltpu.sync_copy(acc.at[stripe], out_hbm.at[core, stripe, :])


def _phase_c3(hc, ex, src_i, dst_i, zeros_acc):
    mesh = plsc.VectorSubcoreMesh(
        core_axis_name="c", subcore_axis_name="s",
        num_cores=NC, num_subcores=NS)
    fn = pl.kernel(
        _phase_c3_body,
        out_type=jax.ShapeDtypeStruct((NC, ACC_R, 128), _f32),
        mesh=mesh,
        compiler_params=pltpu.CompilerParams(needs_layout_passes=False),
        scratch_types=[
            pltpu.VMEM_SHARED((ACC_R, 128), _f32),
            pltpu.VMEM((CB, 128), _f32),
            pltpu.VMEM((CB,), _i32),
            pltpu.VMEM((CB,), _i32),
            pltpu.VMEM((CB,), _f32),
            pltpu.SemaphoreType.DMA,
        ],
    )
    return fn(hc, ex, src_i, dst_i, zeros_acc)


def _k3_body(a_ref, s_ref, b_ref, o_ref):
    rinv = 1.0 / (s_ref[...][:, 0:1] + 1e-16)
    expand = jnp.dot(rinv, jnp.ones((1, OUT), _f32),
                     preferred_element_type=_f32,
                     precision=lax.Precision.HIGHEST)
    o_ref[...] = (a_ref[...][0] + a_ref[...][1]) * expand + b_ref[...]


def _k3(acc3p, s3_comb, b3):
    return pl.pallas_call(
        _k3_body,
        grid=(NBLK,),
        in_specs=[
            pl.BlockSpec((NC, BM, 128), lambda i: (0, i, 0)),
            pl.BlockSpec((BM, HEADS), lambda i: (i, 0)),
            pl.BlockSpec((1, OUT), lambda i: (0, 0)),
        ],
        out_specs=pl.BlockSpec((BM, OUT), lambda i: (i, 0)),
        out_shape=jax.ShapeDtypeStruct((N, OUT), _f32),
    )(acc3p, s3_comb, b3.reshape(1, OUT))


def _proj_mats(a_src, a_dst):
    eye = jnp.eye(HEADS, dtype=_f32)
    pas = (eye[:, :, None] * a_src[None, :, :]).reshape(HEADS, F)
    pad = (eye[:, :, None] * a_dst[None, :, :]).reshape(HEADS, F)
    return pas, pad


def _edge_arrays(edge_index):
    src = edge_index[0].astype(_i32)
    dst = edge_index[1].astype(_i32)
    loop = jnp.arange(N, dtype=_i32)
    pad = EP - (E + N)
    src_i = jnp.concatenate([src, loop, jnp.zeros((pad,), _i32)])
    dst_i = jnp.concatenate([dst, loop, jnp.full((pad,), N, _i32)])
    return src_i, dst_i


def _combine_s(s_hbm):
    s4 = s_hbm.reshape(4, 8, 2, NPD).sum(axis=1)
    return s4.reshape(8, NPD)[:, :N].T


def _erep():
    eye = jnp.eye(HEADS, dtype=_f32)
    return (eye[:, :, None] * jnp.ones((1, HEADS, HID), _f32)).reshape(
        HEADS, F)


def kernel(x, edge_index, W1, a_src1, a_dst1, b1, g1, be1,
           W2, a_src2, a_dst2, b2, g2, be2, W3, a_src3, a_dst3, b3):
    src_i, dst_i = _edge_arrays(edge_index)
    zeros_acc = jnp.zeros((ACC_R, 128), _f32)
    erep = _erep()

    pas1, pad1 = _proj_mats(a_src1, a_dst1)
    hcs1, as1, ad1, c1 = _k1(x, W1, pas1, pad1)
    *exs1, s1 = _phase_b(as1.T.reshape(4, 2, N), ad1.T.reshape(4, 2, N),
                         c1.reshape(64), src_i, dst_i)
    acc1 = _phase_c(hcs1, exs1, src_i, dst_i, zeros_acc)
    s1c = _combine_s(s1)

    pas2, pad2 = _proj_mats(a_src2, a_dst2)
    hcs2, as2, ad2, c2 = _k2(acc1, s1c, b1, g1, be1, W2, pas2, pad2,
                             erep, nout=4)
    *exs2, s2 = _phase_b(as2.T.reshape(4, 2, N), ad2.T.reshape(4, 2, N),
                         c2.reshape(64), src_i, dst_i)
    acc2 = _phase_c(hcs2, exs2, src_i, dst_i, zeros_acc)
    s2c = _combine_s(s2)

    pas3 = jnp.pad(a_src3.astype(_f32), ((0, HEADS - 1), (0, 0)))
    pad3 = jnp.pad(a_dst3.astype(_f32), ((0, HEADS - 1), (0, 0)))
    hcs3, as3, ad3, c3 = _k2(acc2, s2c, b2, g2, be2, W3, pas3, pad3,
                             erep, nout=1)
    ex3, s3 = _phase_b3(as3[:, 0], ad3[:, 0], c3.reshape(64),
                        src_i, dst_i)
    acc3p = _phase_c3(hcs3[0], ex3, src_i, dst_i, zeros_acc)
    s3c = s3.sum(axis=0)[:N]
    s3c8 = jnp.broadcast_to(s3c[:, None], (N, HEADS))
    return _k3(acc3p, s3c8, b3)


# speedup vs baseline: 35.2528x; 1.1082x over previous
import functools

import jax
import jax.numpy as jnp
import numpy as np
from jax import lax
from jax.experimental import pallas as pl
from jax.experimental.pallas import tpu as pltpu
from jax.experimental.pallas import tpu_sc as plsc

N = 10000
E = 320000
D_IN = 128
HID = 64
HEADS = 8
OUT = 128
F = HEADS * HID

NC, NS, L = 2, 16, 16
NW = NC * NS

EP = 330240
NPD = 10016
ACC_R = 10240
STRIPE = ACC_R // NS

BM = 1000
NBLK = N // BM

BSL = EP // 8
BB = 4128

CSL = EP // 16
CB = 160
CNB = CSL // CB
CB3 = 80

_f32 = jnp.float32
_i32 = jnp.int32


def _lrelu(x):
    return jnp.where(x > 0, x, 0.2 * x)


def _k1_body(x_ref, w_ref, pas_ref, pad_ref,
             hc0, hc1, hc2, hc3, as_ref, ad_ref, ms_ref, md_ref, c_ref):
    i = pl.program_id(0)
    h = jnp.dot(x_ref[...], w_ref[...], preferred_element_type=_f32)
    for k, hc in enumerate((hc0, hc1, hc2, hc3)):
        hc[...] = h[:, k * 128:(k + 1) * 128]
    dn = (((1,), (1,)), ((), ()))
    asb = lax.dot_general(h, pas_ref[...], dn, preferred_element_type=_f32,
                          precision=lax.Precision.HIGHEST)
    adb = lax.dot_general(h, pad_ref[...], dn, preferred_element_type=_f32,
                          precision=lax.Precision.HIGHEST)
    as_ref[...] = asb
    ad_ref[...] = adb

    @pl.when(i == 0)
    def _():
        ms_ref[...] = jnp.full((HEADS, HEADS), -1e30, _f32)
        md_ref[...] = jnp.full((HEADS, HEADS), -1e30, _f32)

    ms_ref[...] = jnp.maximum(
        ms_ref[...], jnp.max(asb, axis=0, keepdims=True))
    md_ref[...] = jnp.maximum(
        md_ref[...], jnp.max(adb, axis=0, keepdims=True))

    @pl.when(i == NBLK - 1)
    def _():
        c_ref[...] = _lrelu(ms_ref[...] + md_ref[...])


def _k1(x, w, pas, pad):
    kdim = x.shape[1]
    outs = (
        [jax.ShapeDtypeStruct((N, 128), _f32) for _ in range(4)]
        + [jax.ShapeDtypeStruct((N, HEADS), _f32) for _ in range(2)]
        + [jax.ShapeDtypeStruct((HEADS, HEADS), _f32) for _ in range(3)]
    )
    res = pl.pallas_call(
        _k1_body,
        grid=(NBLK,),
        in_specs=[
            pl.BlockSpec((BM, kdim), lambda i: (i, 0)),
            pl.BlockSpec((kdim, F), lambda i: (0, 0)),
            pl.BlockSpec((HEADS, F), lambda i: (0, 0)),
            pl.BlockSpec((HEADS, F), lambda i: (0, 0)),
        ],
        out_specs=(
            [pl.BlockSpec((BM, 128), lambda i: (i, 0)) for _ in range(4)]
            + [pl.BlockSpec((BM, HEADS), lambda i: (i, 0)) for _ in range(2)]
            + [pl.BlockSpec((HEADS, HEADS), lambda i: (0, 0)) for _ in range(3)]
        ),
        out_shape=outs,
    )(x, w, pas, pad)
    return res[0:4], res[4], res[5], res[8]


def _phase_b_body(asT, adT, cflat, src_hbm, dst_hbm,
                  ex0, ex1, ex2, ex3, s_hbm,
                  tas0, tas1, tad0, tad1, cbuf, srcb, dstb,
                  exb0, exb1, sP0, sP1):
    core = lax.axis_index("c")
    sub = lax.axis_index("s")
    t = core * NS + sub
    chunk = t // 8
    sl = t % 8

    def _z(k, _):
        z = jnp.zeros((L,), _f32)
        zs = pl.ds(k * L, L)
        sP0[zs] = z
        sP1[zs] = z
        return 0
    lax.fori_loop(0, NPD // L, _z, 0)

    for ck, ex_out in enumerate((ex0, ex1, ex2, ex3)):
        @pl.when(chunk == ck)
        def _(ck=ck, ex_out=ex_out):
            pltpu.sync_copy(asT.at[ck, 0], tas0)
            pltpu.sync_copy(asT.at[ck, 1], tas1)
            pltpu.sync_copy(adT.at[ck, 0], tad0)
            pltpu.sync_copy(adT.at[ck, 1], tad1)
            pltpu.sync_copy(cflat.at[pl.ds(0, L)], cbuf)
            cv = cbuf[pl.ds(0, L)]
            c0 = jnp.broadcast_to(cv[2 * ck], (L,))
            c1 = jnp.broadcast_to(cv[2 * ck + 1], (L,))
            base0 = sl * BSL

            def _batch(bi, _):
                eb = pl.multiple_of(base0 + bi * BB, 8)
                pltpu.sync_copy(src_hbm.at[pl.ds(eb, BB)], srcb)
                pltpu.sync_copy(dst_hbm.at[pl.ds(eb, BB)], dstb)

                def _vec(k, _):
                    ks = pl.ds(k * L, L)
                    srcv = srcb[ks]
                    dstv = dstb[ks]
                    for tas, tad, cc, exb, sP in (
                            (tas0, tad0, c0, exb0, sP0),
                            (tas1, tad1, c1, exb1, sP1)):
                        va = plsc.load_gather(tas, [srcv])
                        vd = plsc.load_gather(tad, [dstv])
                        exv = jnp.exp(_lrelu(va + vd) - cc)
                        exb[ks] = exv
                        plsc.addupdate_scatter(sP, [dstv], exv)
                    return 0
                lax.fori_loop(0, BB // L, _vec, 0)
                for hl, exb in ((0, exb0), (1, exb1)):
                    xo = pl.multiple_of((sl * 2 + hl) * BSL + bi * BB, 8)
                    pltpu.sync_copy(exb, ex_out.at[pl.ds(xo, BB)])
                return 0
            lax.fori_loop(0, BSL // BB, _batch, 0)

    pltpu.sync_copy(sP0, s_hbm.at[t, 0])
    pltpu.sync_copy(sP1, s_hbm.at[t, 1])


def _phase_b(asT, adT, cflat, src_i, dst_i):
    mesh = plsc.VectorSubcoreMesh(
        core_axis_name="c", subcore_axis_name="s",
        num_cores=NC, num_subcores=NS)
    outs = (
        [jax.ShapeDtypeStruct((2 * EP,), _f32) for _ in range(4)]
        + [jax.ShapeDtypeStruct((NW, 2, NPD), _f32)]
    )
    fn = pl.kernel(
        _phase_b_body,
        out_type=outs,
        mesh=mesh,
        compiler_params=pltpu.CompilerParams(needs_layout_passes=False),
        scratch_types=[
            pltpu.VMEM((N,), _f32),
            pltpu.VMEM((N,), _f32),
            pltpu.VMEM((N,), _f32),
            pltpu.VMEM((N,), _f32),
            pltpu.VMEM((L,), _f32),
            pltpu.VMEM((BB,), _i32),
            pltpu.VMEM((BB,), _i32),
            pltpu.VMEM((BB,), _f32),
            pltpu.VMEM((BB,), _f32),
            pltpu.VMEM((NPD,), _f32),
            pltpu.VMEM((NPD,), _f32),
        ],
    )
    return fn(asT, adT, cflat, src_i, dst_i)


def _scale_rows_2h(rows, exb0, exb1):
    def _edge16(k, _):
        ks = pl.ds(k * L, L)
        ex0v = exb0[ks]
        ex1v = exb1[ks]
        for i in range(L):
            e = k * L + i
            x0 = jnp.broadcast_to(ex0v[i], (L,))
            x1 = jnp.broadcast_to(ex1v[i], (L,))
            for j in range(8):
                js = pl.ds(j * L, L)
                v = rows[e, js]
                rows[e, js] = v * (x0 if j < 4 else x1)
        return 0
    lax.fori_loop(0, rows.shape[0] // L, _edge16, 0)


def _phase_c_body(hc0, hc1, hc2, hc3, ex0, ex1, ex2, ex3,
                  src_hbm, dst_hbm, zeros_hbm, out_hbm,
                  acc, rowsA, srcbA, dstbA, ex0A, ex1A, semA,
                  rowsB, srcbB, dstbB, ex0B, ex1B, semB):
    core = lax.axis_index("c")
    sub = lax.axis_index("s")
    stripe = pl.ds(pl.multiple_of(sub * STRIPE, STRIPE), STRIPE)
    hcs = (hc0, hc1, hc2, hc3)
    exs = (ex0, ex1, ex2, ex3)
    bufA = (rowsA, srcbA, dstbA, ex0A, ex1A, semA)
    bufB = (rowsB, srcbB, dstbB, ex0B, ex1B, semB)

    for p in range(2):
        pltpu.sync_copy(zeros_hbm.at[stripe], acc.at[stripe])
        plsc.subcore_barrier()

        for ck in range(4):
            if ck % 2 != p:
                continue

            @pl.when(core == ck // 2)
            def _(ck=ck):
                hc = hcs[ck]
                ex = exs[ck]
                base0 = sub * CSL
                bsl = sub // 2
                boff = (sub % 2) * CSL

                def _prefetch(bi, buf):
                    rows, srcb, dstb, exb0, exb1, sem = buf
                    eb = pl.multiple_of(base0 + bi * CB, 8)
                    pltpu.sync_copy(src_hbm.at[pl.ds(eb, CB)], srcb)
                    pltpu.sync_copy(dst_hbm.at[pl.ds(eb, CB)], dstb)
                    off = boff + bi * CB
                    for hl, exb in ((0, exb0), (1, exb1)):
                        xo = pl.multiple_of(
                            (bsl * 2 + hl) * BSL + off, 8)
                        pltpu.sync_copy(ex.at[pl.ds(xo, CB)], exb)
                    pltpu.async_copy(hc.at[srcb], rows, sem)

                def _process(buf):
                    rows, srcb, dstb, exb0, exb1, sem = buf
                    pltpu.make_async_copy(hc.at[srcb], rows, sem).wait()
                    _scale_rows_2h(rows, exb0, exb1)
                    pltpu.sync_copy(rows, acc.at[dstb], add=True)

                _prefetch(0, bufA)

                def _body2(m, _):
                    _prefetch(2 * m + 1, bufB)
                    _process(bufA)

                    @pl.when(2 * m + 2 < CNB)
                    def _():
                        _prefetch(2 * m + 2, bufA)
                    _process(bufB)
                    return 0
                lax.fori_loop(0, CNB // 2, _body2, 0)
                if CNB % 2 == 1:
                    _process(bufA)

        plsc.subcore_barrier()
        col = pl.multiple_of((2 * core + p) * 128, 128)
        pltpu.sync_copy(
            acc.at[stripe], out_hbm.at[stripe, pl.ds(col, 128)])
        plsc.subcore_barrier()


def _phase_c(hcs, exs, src_i, dst_i, zeros_acc):
    mesh = plsc.VectorSubcoreMesh(
        core_axis_name="c", subcore_axis_name="s",
        num_cores=NC, num_subcores=NS)
    dbuf = [
        pltpu.VMEM((CB, 128), _f32),
        pltpu.VMEM((CB,), _i32),
        pltpu.VMEM((CB,), _i32),
        pltpu.VMEM((CB,), _f32),
        pltpu.VMEM((CB,), _f32),
        pltpu.SemaphoreType.DMA,
    ]
    fn = pl.kernel(
        _phase_c_body,
        out_type=jax.ShapeDtypeStruct((ACC_R, F), _f32),
        mesh=mesh,
        compiler_params=pltpu.CompilerParams(needs_layout_passes=False),
        scratch_types=[pltpu.VMEM_SHARED((ACC_R, 128), _f32)] + dbuf + dbuf,
    )
    return fn(*hcs, *exs, src_i, dst_i, zeros_acc)


def _k2_body(nout, acc_ref, s_ref, b_ref, g_ref, be_ref, w_ref,
             pas_ref, pad_ref, erep_ref, *out_refs):
    p = pl.program_id(0)
    i = pl.program_id(1)
    hc_refs = out_refs[:nout]
    as_ref, ad_ref, ms_ref, md_ref, c_ref, ssum_ref, ssq_ref = out_refs[nout:]

    rinv = 1.0 / (s_ref[...] + 1e-16)
    expand = jnp.dot(rinv, erep_ref[...], preferred_element_type=_f32,
                     precision=lax.Precision.HIGHEST)
    z = acc_ref[...] * expand + b_ref[...]

    @pl.when(jnp.logical_and(p == 0, i == 0))
    def _():
        ssum_ref[...] = jnp.zeros((HEADS, F), _f32)
        ssq_ref[...] = jnp.zeros((HEADS, F), _f32)

    @pl.when(p == 0)
    def _():
        ssum_ref[...] += jnp.broadcast_to(
            jnp.sum(z, axis=0, keepdims=True), (HEADS, F))
        ssq_ref[...] += jnp.broadcast_to(
            jnp.sum(z * z, axis=0, keepdims=True), (HEADS, F))

    @pl.when(p == 1)
    def _():
        mean = ssum_ref[...][0:1, :] * (1.0 / N)
        var = ssq_ref[...][0:1, :] * (1.0 / N) - mean * mean
        xn = g_ref[...] * (z - mean) * lax.rsqrt(var + 1e-5) + be_ref[...]
        y = jnp.where(xn > 0, xn, jnp.exp(xn) - 1.0)
        hn = jnp.dot(y, w_ref[...], preferred_element_type=_f32)
        fo = hn.shape[1]
        for k, hc in enumerate(hc_refs):
            hc[...] = hn[:, k * 128:(k + 1) * 128]
        dn = (((1,), (1,)), ((), ()))
        asb = lax.dot_general(hn, pas_ref[...], dn,
                              preferred_element_type=_f32,
                              precision=lax.Precision.HIGHEST)
        adb = lax.dot_general(hn, pad_ref[...], dn,
                              preferred_element_type=_f32,
                              precision=lax.Precision.HIGHEST)
        as_ref[...] = asb
        ad_ref[...] = adb

        @pl.when(i == 0)
        def _():
            ms_ref[...] = jnp.full((HEADS, HEADS), -1e30, _f32)
            md_ref[...] = jnp.full((HEADS, HEADS), -1e30, _f32)

        ms_ref[...] = jnp.maximum(
            ms_ref[...], jnp.max(asb, axis=0, keepdims=True))
        md_ref[...] = jnp.maximum(
            md_ref[...], jnp.max(adb, axis=0, keepdims=True))

        @pl.when(i == NBLK - 1)
        def _():
            c_ref[...] = _lrelu(ms_ref[...] + md_ref[...])


def _k2(acc, s_comb, b, g, be, w, pas, pad, erep, nout):
    fo = w.shape[1]
    outs = (
        [jax.ShapeDtypeStruct((N, 128), _f32) for _ in range(nout)]
        + [jax.ShapeDtypeStruct((N, HEADS), _f32) for _ in range(2)]
        + [jax.ShapeDtypeStruct((HEADS, HEADS), _f32) for _ in range(3)]
        + [jax.ShapeDtypeStruct((HEADS, F), _f32) for _ in range(2)]
    )
    res = pl.pallas_call(
        functools.partial(_k2_body, nout),
        grid=(2, NBLK),
        in_specs=[
            pl.BlockSpec((BM, F), lambda p, i: (i, 0)),
            pl.BlockSpec((BM, HEADS), lambda p, i: (i, 0)),
            pl.BlockSpec((1, F), lambda p, i: (0, 0)),
            pl.BlockSpec((1, F), lambda p, i: (0, 0)),
            pl.BlockSpec((1, F), lambda p, i: (0, 0)),
            pl.BlockSpec((F, fo), lambda p, i: (0, 0)),
            pl.BlockSpec((HEADS, fo), lambda p, i: (0, 0)),
            pl.BlockSpec((HEADS, fo), lambda p, i: (0, 0)),
            pl.BlockSpec((HEADS, F), lambda p, i: (0, 0)),
        ],
        out_specs=(
            [pl.BlockSpec((BM, 128), lambda p, i: (i, 0))
             for _ in range(nout)]
            + [pl.BlockSpec((BM, HEADS), lambda p, i: (i, 0))
               for _ in range(2)]
            + [pl.BlockSpec((HEADS, HEADS), lambda p, i: (0, 0))
               for _ in range(3)]
            + [pl.BlockSpec((HEADS, F), lambda p, i: (0, 0))
               for _ in range(2)]
        ),
        out_shape=outs,
    )(acc[:N], s_comb, b.reshape(1, F), g.reshape(1, F),
      be.reshape(1, F), w, pas, pad, erep)
    return res[:nout], res[nout], res[nout + 1], res[nout + 4]


SL3 = EP // NW


def _phase_b3_body(asf, adf, cflat, src_hbm, dst_hbm, ex_out, s_hbm,
                   tas0, tad0, cbuf, srcb, dstb, exb, sP0):
    core = lax.axis_index("c")
    sub = lax.axis_index("s")
    t = core * NS + sub

    def _z(k, _):
        sP0[pl.ds(k * L, L)] = jnp.zeros((L,), _f32)
        return 0
    lax.fori_loop(0, NPD // L, _z, 0)

    pltpu.sync_copy(asf, tas0)
    pltpu.sync_copy(adf, tad0)
    pltpu.sync_copy(cflat.at[pl.ds(0, L)], cbuf)
    cv = cbuf[pl.ds(0, L)]
    c0 = jnp.broadcast_to(cv[0], (L,))

    eb = pl.multiple_of(t * SL3, 8)
    pltpu.sync_copy(src_hbm.at[pl.ds(eb, SL3)], srcb)
    pltpu.sync_copy(dst_hbm.at[pl.ds(eb, SL3)], dstb)

    def _vec(k, _):
        ks = pl.ds(k * L, L)
        srcv = srcb[ks]
        dstv = dstb[ks]
        va = plsc.load_gather(tas0, [srcv])
        vd = plsc.load_gather(tad0, [dstv])
        exv = jnp.exp(_lrelu(va + vd) - c0)
        exb[ks] = exv
        plsc.addupdate_scatter(sP0, [dstv], exv)
        return 0
    lax.fori_loop(0, SL3 // L, _vec, 0)

    pltpu.sync_copy(exb, ex_out.at[pl.ds(eb, SL3)])
    pltpu.sync_copy(sP0, s_hbm.at[t])


def _phase_b3(asf, adf, cflat, src_i, dst_i):
    mesh = plsc.VectorSubcoreMesh(
        core_axis_name="c", subcore_axis_name="s",
        num_cores=NC, num_subcores=NS)
    outs = [jax.ShapeDtypeStruct((EP,), _f32),
            jax.ShapeDtypeStruct((NW, NPD), _f32)]
    fn = pl.kernel(
        _phase_b3_body,
        out_type=outs,
        mesh=mesh,
        compiler_params=pltpu.CompilerParams(needs_layout_passes=False),
        scratch_types=[
            pltpu.VMEM((N,), _f32),
            pltpu.VMEM((N,), _f32),
            pltpu.VMEM((L,), _f32),
            pltpu.VMEM((SL3,), _i32),
            pltpu.VMEM((SL3,), _i32),
            pltpu.VMEM((SL3,), _f32),
            pltpu.VMEM((NPD,), _f32),
        ],
    )
    return fn(asf, adf, cflat, src_i, dst_i)


CNB3 = SL3 // CB3


def _scale_rows_1h(rows, exb0):
    def _edge16(k, _):
        ks = pl.ds(k * L, L)
        ex0v = exb0[ks]
        for i in range(L):
            e = k * L + i
            x0 = jnp.broadcast_to(ex0v[i], (L,))
            for j in range(8):
                js = pl.ds(j * L, L)
                v = rows[e, js]
                rows[e, js] = v * x0
        return 0
    lax.fori_loop(0, rows.shape[0] // L, _edge16, 0)


def _phase_c3_body(hc, ex, src_hbm, dst_hbm, zeros_hbm, out_hbm,
                   acc, rowsA, srcbA, dstbA, ex0A, semA,
                   rowsB, srcbB, dstbB, ex0B, semB):
    core = lax.axis_index("c")
    sub = lax.axis_index("s")
    t = core * NS + sub
    stripe = pl.ds(pl.multiple_of(sub * STRIPE, STRIPE), STRIPE)
    bufA = (rowsA, srcbA, dstbA, ex0A, semA)
    bufB = (rowsB, srcbB, dstbB, ex0B, semB)

    pltpu.sync_copy(zeros_hbm.at[stripe], acc.at[stripe])
    plsc.subcore_barrier()

    base0 = t * SL3

    def _prefetch(bi, buf):
        rows, srcb, dstb, exb0, sem = buf
        eb = pl.multiple_of(base0 + bi * CB3, 8)
        pltpu.sync_copy(src_hbm.at[pl.ds(eb, CB3)], srcb)
        pltpu.sync_copy(dst_hbm.at[pl.ds(eb, CB3)], dstb)
        pltpu.sync_copy(ex.at[pl.ds(eb, CB3)], exb0)
        pltpu.async_copy(hc.at[srcb], rows, sem)

    def _process(buf):
        rows, srcb, dstb, exb0, sem = buf
        pltpu.make_async_copy(hc.at[srcb], rows, sem).wait()
        _scale_rows_1h(rows, exb0)
        pltpu.sync_copy(rows, acc.at[dstb], add=True)

    _prefetch(0, bufA)

    def _body2(m, _):
        _prefetch(2 * m + 1, bufB)
        _process(bufA)

        @pl.when(2 * m + 2 < CNB3)
        def _():
            _prefetch(2 * m + 2, bufA)
        _process(bufB)
        return 0
    lax.fori_loop(0, CNB3 // 2, _body2, 0)
    if CNB3 % 2 == 1:
        _process(bufA)

    plsc.subcore_barrier()
    pltpu.sync_copy(acc.at[stripe], out_hbm.at[core, stripe, :])


def _phase_c3(hc, ex, src_i, dst_i, zeros_acc):
    mesh = plsc.VectorSubcoreMesh(
        core_axis_name="c", subcore_axis_name="s",
        num_cores=NC, num_subcores=NS)
    dbuf = [
        pltpu.VMEM((CB3, 128), _f32),
        pltpu.VMEM((CB3,), _i32),
        pltpu.VMEM((CB3,), _i32),
        pltpu.VMEM((CB3,), _f32),
        pltpu.SemaphoreType.DMA,
    ]
    fn = pl.kernel(
        _phase_c3_body,
        out_type=jax.ShapeDtypeStruct((NC, ACC_R, 128), _f32),
        mesh=mesh,
        compiler_params=pltpu.CompilerParams(needs_layout_passes=False),
        scratch_types=[pltpu.VMEM_SHARED((ACC_R, 128), _f32)] + dbuf + dbuf,
    )
    return fn(hc, ex, src_i, dst_i, zeros_acc)


def _k3_body(a_ref, s_ref, b_ref, o_ref):
    rinv = 1.0 / (s_ref[...][:, 0:1] + 1e-16)
    expand = jnp.dot(rinv, jnp.ones((1, OUT), _f32),
                     preferred_element_type=_f32,
                     precision=lax.Precision.HIGHEST)
    o_ref[...] = (a_ref[...][0] + a_ref[...][1]) * expand + b_ref[...]


def _k3(acc3p, s3_comb, b3):
    return pl.pallas_call(
        _k3_body,
        grid=(NBLK,),
        in_specs=[
            pl.BlockSpec((NC, BM, 128), lambda i: (0, i, 0)),
            pl.BlockSpec((BM, HEADS), lambda i: (i, 0)),
            pl.BlockSpec((1, OUT), lambda i: (0, 0)),
        ],
        out_specs=pl.BlockSpec((BM, OUT), lambda i: (i, 0)),
        out_shape=jax.ShapeDtypeStruct((N, OUT), _f32),
    )(acc3p, s3_comb, b3.reshape(1, OUT))


def _proj_mats(a_src, a_dst):
    eye = jnp.eye(HEADS, dtype=_f32)
    pas = (eye[:, :, None] * a_src[None, :, :]).reshape(HEADS, F)
    pad = (eye[:, :, None] * a_dst[None, :, :]).reshape(HEADS, F)
    return pas, pad


def _edge_arrays(edge_index):
    src = edge_index[0].astype(_i32)
    dst = edge_index[1].astype(_i32)
    loop = jnp.arange(N, dtype=_i32)
    pad = EP - (E + N)
    src_i = jnp.concatenate([src, loop, jnp.zeros((pad,), _i32)])
    dst_i = jnp.concatenate([dst, loop, jnp.full((pad,), N, _i32)])
    return src_i, dst_i


def _combine_s(s_hbm):
    s4 = s_hbm.reshape(4, 8, 2, NPD).sum(axis=1)
    return s4.reshape(8, NPD)[:, :N].T


def _erep():
    eye = jnp.eye(HEADS, dtype=_f32)
    return (eye[:, :, None] * jnp.ones((1, HEADS, HID), _f32)).reshape(
        HEADS, F)


def kernel(x, edge_index, W1, a_src1, a_dst1, b1, g1, be1,
           W2, a_src2, a_dst2, b2, g2, be2, W3, a_src3, a_dst3, b3):
    src_i, dst_i = _edge_arrays(edge_index)
    zeros_acc = jnp.zeros((ACC_R, 128), _f32)
    erep = _erep()

    pas1, pad1 = _proj_mats(a_src1, a_dst1)
    hcs1, as1, ad1, c1 = _k1(x, W1, pas1, pad1)
    *exs1, s1 = _phase_b(as1.T.reshape(4, 2, N), ad1.T.reshape(4, 2, N),
                         c1.reshape(64), src_i, dst_i)
    acc1 = _phase_c(hcs1, exs1, src_i, dst_i, zeros_acc)
    s1c = _combine_s(s1)

    pas2, pad2 = _proj_mats(a_src2, a_dst2)
    hcs2, as2, ad2, c2 = _k2(acc1, s1c, b1, g1, be1, W2, pas2, pad2,
                             erep, nout=4)
    *exs2, s2 = _phase_b(as2.T.reshape(4, 2, N), ad2.T.reshape(4, 2, N),
                         c2.reshape(64), src_i, dst_i)
    acc2 = _phase_c(hcs2, exs2, src_i, dst_i, zeros_acc)
    s2c = _combine_s(s2)

    pas3 = jnp.pad(a_src3.astype(_f32), ((0, HEADS - 1), (0, 0)))
    pad3 = jnp.pad(a_dst3.astype(_f32), ((0, HEADS - 1), (0, 0)))
    hcs3, as3, ad3, c3 = _k2(acc2, s2c, b2, g2, be2, W3, pas3, pad3,
                             erep, nout=1)
    ex3, s3 = _phase_b3(as3[:, 0], ad3[:, 0], c3.reshape(64),
                        src_i, dst_i)
    acc3p = _phase_c3(hcs3[0], ex3, src_i, dst_i, zeros_acc)
    s3c = s3.sum(axis=0)[:N]
    s3c8 = jnp.broadcast_to(s3c[:, None], (N, HEADS))
    return _k3(acc3p, s3c8, b3)


# speedup vs baseline: 38.5474x; 1.0935x over previous
import functools

import jax
import jax.numpy as jnp
import numpy as np
from jax import lax
from jax.experimental import pallas as pl
from jax.experimental.pallas import tpu as pltpu
from jax.experimental.pallas import tpu_sc as plsc

N = 10000
E = 320000
D_IN = 128
HID = 64
HEADS = 8
OUT = 128
F = HEADS * HID

NC, NS, L = 2, 16, 16
NW = NC * NS

EP = 330240
NPD = 10016
ACC_R = 10240
STRIPE = ACC_R // NS

BM = 1000
NBLK = N // BM

BSL = EP // 8
BB = 4128

CSL = EP // 16
CB = 160
CNB = CSL // CB
CB3 = 120

_f32 = jnp.float32
_i32 = jnp.int32


def _lrelu(x):
    return jnp.where(x > 0, x, 0.2 * x)


def _k1_body(x_ref, w_ref, pas_ref, pad_ref,
             hc0, hc1, hc2, hc3, as_ref, ad_ref, ms_ref, md_ref, c_ref):
    i = pl.program_id(0)
    h = jnp.dot(x_ref[...], w_ref[...], preferred_element_type=_f32)
    for k, hc in enumerate((hc0, hc1, hc2, hc3)):
        hc[...] = h[:, k * 128:(k + 1) * 128]
    dn = (((1,), (1,)), ((), ()))
    asb = lax.dot_general(h, pas_ref[...], dn, preferred_element_type=_f32,
                          precision=lax.Precision.HIGHEST)
    adb = lax.dot_general(h, pad_ref[...], dn, preferred_element_type=_f32,
                          precision=lax.Precision.HIGHEST)
    as_ref[...] = asb
    ad_ref[...] = adb

    @pl.when(i == 0)
    def _():
        ms_ref[...] = jnp.full((HEADS, HEADS), -1e30, _f32)
        md_ref[...] = jnp.full((HEADS, HEADS), -1e30, _f32)

    ms_ref[...] = jnp.maximum(
        ms_ref[...], jnp.max(asb, axis=0, keepdims=True))
    md_ref[...] = jnp.maximum(
        md_ref[...], jnp.max(adb, axis=0, keepdims=True))

    @pl.when(i == NBLK - 1)
    def _():
        c_ref[...] = _lrelu(ms_ref[...] + md_ref[...])


def _k1(x, w, pas, pad):
    kdim = x.shape[1]
    outs = (
        [jax.ShapeDtypeStruct((N, 128), _f32) for _ in range(4)]
        + [jax.ShapeDtypeStruct((N, HEADS), _f32) for _ in range(2)]
        + [jax.ShapeDtypeStruct((HEADS, HEADS), _f32) for _ in range(3)]
    )
    res = pl.pallas_call(
        _k1_body,
        grid=(NBLK,),
        in_specs=[
            pl.BlockSpec((BM, kdim), lambda i: (i, 0)),
            pl.BlockSpec((kdim, F), lambda i: (0, 0)),
            pl.BlockSpec((HEADS, F), lambda i: (0, 0)),
            pl.BlockSpec((HEADS, F), lambda i: (0, 0)),
        ],
        out_specs=(
            [pl.BlockSpec((BM, 128), lambda i: (i, 0)) for _ in range(4)]
            + [pl.BlockSpec((BM, HEADS), lambda i: (i, 0)) for _ in range(2)]
            + [pl.BlockSpec((HEADS, HEADS), lambda i: (0, 0)) for _ in range(3)]
        ),
        out_shape=outs,
    )(x, w, pas, pad)
    return res[0:4], res[4], res[5], res[8]


def _phase_b_body(asT, adT, cflat, src_hbm, dst_hbm,
                  ex0, ex1, ex2, ex3, s_hbm,
                  tas0, tas1, tad0, tad1, cbuf, srcb, dstb,
                  exb0, exb1, sP0, sP1):
    core = lax.axis_index("c")
    sub = lax.axis_index("s")
    t = core * NS + sub
    chunk = t // 8
    sl = t % 8

    def _z(k, _):
        z = jnp.zeros((L,), _f32)
        zs = pl.ds(k * L, L)
        sP0[zs] = z
        sP1[zs] = z
        return 0
    lax.fori_loop(0, NPD // L, _z, 0)

    for ck, ex_out in enumerate((ex0, ex1, ex2, ex3)):
        @pl.when(chunk == ck)
        def _(ck=ck, ex_out=ex_out):
            pltpu.sync_copy(asT.at[ck, 0], tas0)
            pltpu.sync_copy(asT.at[ck, 1], tas1)
            pltpu.sync_copy(adT.at[ck, 0], tad0)
            pltpu.sync_copy(adT.at[ck, 1], tad1)
            pltpu.sync_copy(cflat.at[pl.ds(0, L)], cbuf)
            cv = cbuf[pl.ds(0, L)]
            c0 = jnp.broadcast_to(cv[2 * ck], (L,))
            c1 = jnp.broadcast_to(cv[2 * ck + 1], (L,))
            base0 = sl * BSL

            def _batch(bi, _):
                eb = pl.multiple_of(base0 + bi * BB, 8)
                pltpu.sync_copy(src_hbm.at[pl.ds(eb, BB)], srcb)
                pltpu.sync_copy(dst_hbm.at[pl.ds(eb, BB)], dstb)

                def _vec(k, _):
                    ks = pl.ds(k * L, L)
                    srcv = srcb[ks]
                    dstv = dstb[ks]
                    for tas, tad, cc, exb, sP in (
                            (tas0, tad0, c0, exb0, sP0),
                            (tas1, tad1, c1, exb1, sP1)):
                        va = plsc.load_gather(tas, [srcv])
                        vd = plsc.load_gather(tad, [dstv])
                        exv = jnp.exp(_lrelu(va + vd) - cc)
                        exb[ks] = exv
                        plsc.addupdate_scatter(sP, [dstv], exv)
                    return 0
                lax.fori_loop(0, BB // L, _vec, 0)
                for hl, exb in ((0, exb0), (1, exb1)):
                    xo = pl.multiple_of((sl * 2 + hl) * BSL + bi * BB, 8)
                    pltpu.sync_copy(exb, ex_out.at[pl.ds(xo, BB)])
                return 0
            lax.fori_loop(0, BSL // BB, _batch, 0)

    pltpu.sync_copy(sP0, s_hbm.at[t, 0])
    pltpu.sync_copy(sP1, s_hbm.at[t, 1])


def _phase_b(asT, adT, cflat, src_i, dst_i):
    mesh = plsc.VectorSubcoreMesh(
        core_axis_name="c", subcore_axis_name="s",
        num_cores=NC, num_subcores=NS)
    outs = (
        [jax.ShapeDtypeStruct((2 * EP,), _f32) for _ in range(4)]
        + [jax.ShapeDtypeStruct((NW, 2, NPD), _f32)]
    )
    fn = pl.kernel(
        _phase_b_body,
        out_type=outs,
        mesh=mesh,
        compiler_params=pltpu.CompilerParams(needs_layout_passes=False),
        scratch_types=[
            pltpu.VMEM((N,), _f32),
            pltpu.VMEM((N,), _f32),
            pltpu.VMEM((N,), _f32),
            pltpu.VMEM((N,), _f32),
            pltpu.VMEM((L,), _f32),
            pltpu.VMEM((BB,), _i32),
            pltpu.VMEM((BB,), _i32),
            pltpu.VMEM((BB,), _f32),
            pltpu.VMEM((BB,), _f32),
            pltpu.VMEM((NPD,), _f32),
            pltpu.VMEM((NPD,), _f32),
        ],
    )
    return fn(asT, adT, cflat, src_i, dst_i)


def _scale_rows_2h(rows, exb0, exb1):
    def _edge16(k, _):
        ks = pl.ds(k * L, L)
        ex0v = exb0[ks]
        ex1v = exb1[ks]
        for i in range(L):
            e = k * L + i
            x0 = jnp.broadcast_to(ex0v[i], (L,))
            x1 = jnp.broadcast_to(ex1v[i], (L,))
            for j in range(8):
                js = pl.ds(j * L, L)
                v = rows[e, js]
                rows[e, js] = v * (x0 if j < 4 else x1)
        return 0
    lax.fori_loop(0, rows.shape[0] // L, _edge16, 0)


def _phase_c_body(hc0, hc1, hc2, hc3, ex0, ex1, ex2, ex3,
                  src_hbm, dst_hbm, zeros_hbm, out_hbm,
                  acc, rowsA, srcbA, dstbA, ex0A, ex1A, semA, semSA,
                  rowsB, srcbB, dstbB, ex0B, ex1B, semB, semSB):
    core = lax.axis_index("c")
    sub = lax.axis_index("s")
    stripe = pl.ds(pl.multiple_of(sub * STRIPE, STRIPE), STRIPE)
    hcs = (hc0, hc1, hc2, hc3)
    exs = (ex0, ex1, ex2, ex3)
    bufA = (rowsA, srcbA, dstbA, ex0A, ex1A, semA, semSA)
    bufB = (rowsB, srcbB, dstbB, ex0B, ex1B, semB, semSB)

    for p in range(2):
        pltpu.sync_copy(zeros_hbm.at[stripe], acc.at[stripe])
        plsc.subcore_barrier()

        for ck in range(4):
            if ck % 2 != p:
                continue

            @pl.when(core == ck // 2)
            def _(ck=ck):
                hc = hcs[ck]
                ex = exs[ck]
                base0 = sub * CSL
                bsl = sub // 2
                boff = (sub % 2) * CSL

                def _prefetch(bi, buf):
                    rows, srcb, dstb, exb0, exb1, sem, semS = buf
                    eb = pl.multiple_of(base0 + bi * CB, 8)
                    pltpu.sync_copy(src_hbm.at[pl.ds(eb, CB)], srcb)
                    pltpu.sync_copy(dst_hbm.at[pl.ds(eb, CB)], dstb)
                    off = boff + bi * CB
                    for hl, exb in ((0, exb0), (1, exb1)):
                        xo = pl.multiple_of(
                            (bsl * 2 + hl) * BSL + off, 8)
                        pltpu.sync_copy(ex.at[pl.ds(xo, CB)], exb)
                    pltpu.async_copy(hc.at[srcb], rows, sem)

                def _process(buf):
                    rows, srcb, dstb, exb0, exb1, sem, semS = buf
                    pltpu.make_async_copy(hc.at[srcb], rows, sem).wait()
                    _scale_rows_2h(rows, exb0, exb1)
                    pltpu.async_copy(rows, acc.at[dstb], semS, add=True)

                def _drain(buf):
                    rows, srcb, dstb, exb0, exb1, sem, semS = buf
                    pltpu.make_async_copy(
                        rows, acc.at[dstb], semS).wait()

                _prefetch(0, bufA)
                _prefetch(1, bufB)

                def _body2(m, _):
                    _process(bufA)
                    _process(bufB)
                    _drain(bufA)
                    _prefetch(2 * m + 2, bufA)

                    @pl.when(2 * m + 3 < CNB)
                    def _():
                        _drain(bufB)
                        _prefetch(2 * m + 3, bufB)
                    return 0
                lax.fori_loop(0, CNB // 2, _body2, 0)
                if CNB % 2 == 1:
                    _process(bufA)
                    _drain(bufA)
                _drain(bufB)

        plsc.subcore_barrier()
        col = pl.multiple_of((2 * core + p) * 128, 128)
        pltpu.sync_copy(
            acc.at[stripe], out_hbm.at[stripe, pl.ds(col, 128)])
        plsc.subcore_barrier()


def _phase_c(hcs, exs, src_i, dst_i, zeros_acc):
    mesh = plsc.VectorSubcoreMesh(
        core_axis_name="c", subcore_axis_name="s",
        num_cores=NC, num_subcores=NS)
    dbuf = [
        pltpu.VMEM((CB, 128), _f32),
        pltpu.VMEM((CB,), _i32),
        pltpu.VMEM((CB,), _i32),
        pltpu.VMEM((CB,), _f32),
        pltpu.VMEM((CB,), _f32),
        pltpu.SemaphoreType.DMA,
        pltpu.SemaphoreType.DMA,
    ]
    fn = pl.kernel(
        _phase_c_body,
        out_type=jax.ShapeDtypeStruct((ACC_R, F), _f32),
        mesh=mesh,
        compiler_params=pltpu.CompilerParams(needs_layout_passes=False),
        scratch_types=[pltpu.VMEM_SHARED((ACC_R, 128), _f32)] + dbuf + dbuf,
    )
    return fn(*hcs, *exs, src_i, dst_i, zeros_acc)


def _k2_body(nout, acc_ref, s_ref, b_ref, g_ref, be_ref, w_ref,
             pas_ref, pad_ref, erep_ref, *out_refs):
    p = pl.program_id(0)
    i = pl.program_id(1)
    hc_refs = out_refs[:nout]
    as_ref, ad_ref, ms_ref, md_ref, c_ref, ssum_ref, ssq_ref = out_refs[nout:]

    rinv = 1.0 / (s_ref[...] + 1e-16)
    expand = jnp.dot(rinv, erep_ref[...], preferred_element_type=_f32,
                     precision=lax.Precision.HIGHEST)
    z = acc_ref[...] * expand + b_ref[...]

    @pl.when(jnp.logical_and(p == 0, i == 0))
    def _():
        ssum_ref[...] = jnp.zeros((HEADS, F), _f32)
        ssq_ref[...] = jnp.zeros((HEADS, F), _f32)

    @pl.when(p == 0)
    def _():
        ssum_ref[...] += jnp.broadcast_to(
            jnp.sum(z, axis=0, keepdims=True), (HEADS, F))
        ssq_ref[...] += jnp.broadcast_to(
            jnp.sum(z * z, axis=0, keepdims=True), (HEADS, F))

    @pl.when(p == 1)
    def _():
        mean = ssum_ref[...][0:1, :] * (1.0 / N)
        var = ssq_ref[...][0:1, :] * (1.0 / N) - mean * mean
        xn = g_ref[...] * (z - mean) * lax.rsqrt(var + 1e-5) + be_ref[...]
        y = jnp.where(xn > 0, xn, jnp.exp(xn) - 1.0)
        hn = jnp.dot(y, w_ref[...], preferred_element_type=_f32)
        fo = hn.shape[1]
        for k, hc in enumerate(hc_refs):
            hc[...] = hn[:, k * 128:(k + 1) * 128]
        dn = (((1,), (1,)), ((), ()))
        asb = lax.dot_general(hn, pas_ref[...], dn,
                              preferred_element_type=_f32,
                              precision=lax.Precision.HIGHEST)
        adb = lax.dot_general(hn, pad_ref[...], dn,
                              preferred_element_type=_f32,
                              precision=lax.Precision.HIGHEST)
        as_ref[...] = asb
        ad_ref[...] = adb

        @pl.when(i == 0)
        def _():
            ms_ref[...] = jnp.full((HEADS, HEADS), -1e30, _f32)
            md_ref[...] = jnp.full((HEADS, HEADS), -1e30, _f32)

        ms_ref[...] = jnp.maximum(
            ms_ref[...], jnp.max(asb, axis=0, keepdims=True))
        md_ref[...] = jnp.maximum(
            md_ref[...], jnp.max(adb, axis=0, keepdims=True))

        @pl.when(i == NBLK - 1)
        def _():
            c_ref[...] = _lrelu(ms_ref[...] + md_ref[...])


def _k2(acc, s_comb, b, g, be, w, pas, pad, erep, nout):
    fo = w.shape[1]
    outs = (
        [jax.ShapeDtypeStruct((N, 128), _f32) for _ in range(nout)]
        + [jax.ShapeDtypeStruct((N, HEADS), _f32) for _ in range(2)]
        + [jax.ShapeDtypeStruct((HEADS, HEADS), _f32) for _ in range(3)]
        + [jax.ShapeDtypeStruct((HEADS, F), _f32) for _ in range(2)]
    )
    res = pl.pallas_call(
        functools.partial(_k2_body, nout),
        grid=(2, NBLK),
        in_specs=[
            pl.BlockSpec((BM, F), lambda p, i: (i, 0)),
            pl.BlockSpec((BM, HEADS), lambda p, i: (i, 0)),
            pl.BlockSpec((1, F), lambda p, i: (0, 0)),
            pl.BlockSpec((1, F), lambda p, i: (0, 0)),
            pl.BlockSpec((1, F), lambda p, i: (0, 0)),
            pl.BlockSpec((F, fo), lambda p, i: (0, 0)),
            pl.BlockSpec((HEADS, fo), lambda p, i: (0, 0)),
            pl.BlockSpec((HEADS, fo), lambda p, i: (0, 0)),
            pl.BlockSpec((HEADS, F), lambda p, i: (0, 0)),
        ],
        out_specs=(
            [pl.BlockSpec((BM, 128), lambda p, i: (i, 0))
             for _ in range(nout)]
            + [pl.BlockSpec((BM, HEADS), lambda p, i: (i, 0))
               for _ in range(2)]
            + [pl.BlockSpec((HEADS, HEADS), lambda p, i: (0, 0))
               for _ in range(3)]
            + [pl.BlockSpec((HEADS, F), lambda p, i: (0, 0))
               for _ in range(2)]
        ),
        out_shape=outs,
    )(acc[:N], s_comb, b.reshape(1, F), g.reshape(1, F),
      be.reshape(1, F), w, pas, pad, erep)
    return res[:nout], res[nout], res[nout + 1], res[nout + 4]


SL3 = EP // NW


def _phase_b3_body(asf, adf, cflat, src_hbm, dst_hbm, ex_out, s_hbm,
                   tas0, tad0, cbuf, srcb, dstb, exb, sP0):
    core = lax.axis_index("c")
    sub = lax.axis_index("s")
    t = core * NS + sub

    def _z(k, _):
        sP0[pl.ds(k * L, L)] = jnp.zeros((L,), _f32)
        return 0
    lax.fori_loop(0, NPD // L, _z, 0)

    pltpu.sync_copy(asf, tas0)
    pltpu.sync_copy(adf, tad0)
    pltpu.sync_copy(cflat.at[pl.ds(0, L)], cbuf)
    cv = cbuf[pl.ds(0, L)]
    c0 = jnp.broadcast_to(cv[0], (L,))

    eb = pl.multiple_of(t * SL3, 8)
    pltpu.sync_copy(src_hbm.at[pl.ds(eb, SL3)], srcb)
    pltpu.sync_copy(dst_hbm.at[pl.ds(eb, SL3)], dstb)

    def _vec(k, _):
        ks = pl.ds(k * L, L)
        srcv = srcb[ks]
        dstv = dstb[ks]
        va = plsc.load_gather(tas0, [srcv])
        vd = plsc.load_gather(tad0, [dstv])
        exv = jnp.exp(_lrelu(va + vd) - c0)
        exb[ks] = exv
        plsc.addupdate_scatter(sP0, [dstv], exv)
        return 0
    lax.fori_loop(0, SL3 // L, _vec, 0)

    pltpu.sync_copy(exb, ex_out.at[pl.ds(eb, SL3)])
    pltpu.sync_copy(sP0, s_hbm.at[t])


def _phase_b3(asf, adf, cflat, src_i, dst_i):
    mesh = plsc.VectorSubcoreMesh(
        core_axis_name="c", subcore_axis_name="s",
        num_cores=NC, num_subcores=NS)
    outs = [jax.ShapeDtypeStruct((EP,), _f32),
            jax.ShapeDtypeStruct((NW, NPD), _f32)]
    fn = pl.kernel(
        _phase_b3_body,
        out_type=outs,
        mesh=mesh,
        compiler_params=pltpu.CompilerParams(needs_layout_passes=False),
        scratch_types=[
            pltpu.VMEM((N,), _f32),
            pltpu.VMEM((N,), _f32),
            pltpu.VMEM((L,), _f32),
            pltpu.VMEM((SL3,), _i32),
            pltpu.VMEM((SL3,), _i32),
            pltpu.VMEM((SL3,), _f32),
            pltpu.VMEM((NPD,), _f32),
        ],
    )
    return fn(asf, adf, cflat, src_i, dst_i)


CNB3 = SL3 // CB3


def _scale_rows_1h(rows, exb0):
    def _edge16(k, _):
        ks = pl.ds(k * L, L)
        ex0v = exb0[ks]
        for i in range(L):
            e = k * L + i
            x0 = jnp.broadcast_to(ex0v[i], (L,))
            for j in range(8):
                js = pl.ds(j * L, L)
                v = rows[e, js]
                rows[e, js] = v * x0
        return 0
    lax.fori_loop(0, rows.shape[0] // L, _edge16, 0)


def _phase_c3_body(hc, ex, src_hbm, dst_hbm, zeros_hbm, out_hbm,
                   acc, rowsA, srcbA, dstbA, ex0A, semA, semSA,
                   rowsB, srcbB, dstbB, ex0B, semB, semSB):
    core = lax.axis_index("c")
    sub = lax.axis_index("s")
    t = core * NS + sub
    stripe = pl.ds(pl.multiple_of(sub * STRIPE, STRIPE), STRIPE)
    bufA = (rowsA, srcbA, dstbA, ex0A, semA, semSA)
    bufB = (rowsB, srcbB, dstbB, ex0B, semB, semSB)

    pltpu.sync_copy(zeros_hbm.at[stripe], acc.at[stripe])
    plsc.subcore_barrier()

    base0 = t * SL3

    def _prefetch(bi, buf):
        rows, srcb, dstb, exb0, sem, semS = buf
        eb = pl.multiple_of(base0 + bi * CB3, 8)
        pltpu.sync_copy(src_hbm.at[pl.ds(eb, CB3)], srcb)
        pltpu.sync_copy(dst_hbm.at[pl.ds(eb, CB3)], dstb)
        pltpu.sync_copy(ex.at[pl.ds(eb, CB3)], exb0)
        pltpu.async_copy(hc.at[srcb], rows, sem)

    def _process(buf):
        rows, srcb, dstb, exb0, sem, semS = buf
        pltpu.make_async_copy(hc.at[srcb], rows, sem).wait()
        _scale_rows_1h(rows, exb0)
        pltpu.async_copy(rows, acc.at[dstb], semS, add=True)

    def _drain(buf):
        rows, srcb, dstb, exb0, sem, semS = buf
        pltpu.make_async_copy(rows, acc.at[dstb], semS).wait()

    _prefetch(0, bufA)
    _prefetch(1, bufB)

    def _body2(m, _):
        _process(bufA)
        _process(bufB)
        _drain(bufA)

        @pl.when(2 * m + 2 < CNB3)
        def _():
            _prefetch(2 * m + 2, bufA)

        @pl.when(2 * m + 3 < CNB3)
        def _():
            _drain(bufB)
            _prefetch(2 * m + 3, bufB)
        return 0
    lax.fori_loop(0, CNB3 // 2, _body2, 0)
    _drain(bufB)

    plsc.subcore_barrier()
    pltpu.sync_copy(acc.at[stripe], out_hbm.at[core, stripe, :])


def _phase_c3(hc, ex, src_i, dst_i, zeros_acc):
    mesh = plsc.VectorSubcoreMesh(
        core_axis_name="c", subcore_axis_name="s",
        num_cores=NC, num_subcores=NS)
    dbuf = [
        pltpu.VMEM((CB3, 128), _f32),
        pltpu.VMEM((CB3,), _i32),
        pltpu.VMEM((CB3,), _i32),
        pltpu.VMEM((CB3,), _f32),
        pltpu.SemaphoreType.DMA,
        pltpu.SemaphoreType.DMA,
    ]
    fn = pl.kernel(
        _phase_c3_body,
        out_type=jax.ShapeDtypeStruct((NC, ACC_R, 128), _f32),
        mesh=mesh,
        compiler_params=pltpu.CompilerParams(needs_layout_passes=False),
        scratch_types=[pltpu.VMEM_SHARED((ACC_R, 128), _f32)] + dbuf + dbuf,
    )
    return fn(hc, ex, src_i, dst_i, zeros_acc)


def _k3_body(a_ref, s_ref, b_ref, o_ref):
    rinv = 1.0 / (s_ref[...][:, 0:1] + 1e-16)
    expand = jnp.dot(rinv, jnp.ones((1, OUT), _f32),
                     preferred_element_type=_f32,
                     precision=lax.Precision.HIGHEST)
    o_ref[...] = (a_ref[...][0] + a_ref[...][1]) * expand + b_ref[...]


def _k3(acc3p, s3_comb, b3):
    return pl.pallas_call(
        _k3_body,
        grid=(NBLK,),
        in_specs=[
            pl.BlockSpec((NC, BM, 128), lambda i: (0, i, 0)),
            pl.BlockSpec((BM, HEADS), lambda i: (i, 0)),
            pl.BlockSpec((1, OUT), lambda i: (0, 0)),
        ],
        out_specs=pl.BlockSpec((BM, OUT), lambda i: (i, 0)),
        out_shape=jax.ShapeDtypeStruct((N, OUT), _f32),
    )(acc3p, s3_comb, b3.reshape(1, OUT))


def _proj_mats(a_src, a_dst):
    eye = jnp.eye(HEADS, dtype=_f32)
    pas = (eye[:, :, None] * a_src[None, :, :]).reshape(HEADS, F)
    pad = (eye[:, :, None] * a_dst[None, :, :]).reshape(HEADS, F)
    return pas, pad


def _edge_arrays(edge_index):
    src = edge_index[0].astype(_i32)
    dst = edge_index[1].astype(_i32)
    loop = jnp.arange(N, dtype=_i32)
    pad = EP - (E + N)
    src_i = jnp.concatenate([src, loop, jnp.zeros((pad,), _i32)])
    dst_i = jnp.concatenate([dst, loop, jnp.full((pad,), N, _i32)])
    return src_i, dst_i


def _combine_s(s_hbm):
    s4 = s_hbm.reshape(4, 8, 2, NPD).sum(axis=1)
    return s4.reshape(8, NPD)[:, :N].T


def _erep():
    eye = jnp.eye(HEADS, dtype=_f32)
    return (eye[:, :, None] * jnp.ones((1, HEADS, HID), _f32)).reshape(
        HEADS, F)


def kernel(x, edge_index, W1, a_src1, a_dst1, b1, g1, be1,
           W2, a_src2, a_dst2, b2, g2, be2, W3, a_src3, a_dst3, b3):
    src_i, dst_i = _edge_arrays(edge_index)
    zeros_acc = jnp.zeros((ACC_R, 128), _f32)
    erep = _erep()

    pas1, pad1 = _proj_mats(a_src1, a_dst1)
    hcs1, as1, ad1, c1 = _k1(x, W1, pas1, pad1)
    *exs1, s1 = _phase_b(as1.T.reshape(4, 2, N), ad1.T.reshape(4, 2, N),
                         c1.reshape(64), src_i, dst_i)
    acc1 = _phase_c(hcs1, exs1, src_i, dst_i, zeros_acc)
    s1c = _combine_s(s1)

    pas2, pad2 = _proj_mats(a_src2, a_dst2)
    hcs2, as2, ad2, c2 = _k2(acc1, s1c, b1, g1, be1, W2, pas2, pad2,
                             erep, nout=4)
    *exs2, s2 = _phase_b(as2.T.reshape(4, 2, N), ad2.T.reshape(4, 2, N),
                         c2.reshape(64), src_i, dst_i)
    acc2 = _phase_c(hcs2, exs2, src_i, dst_i, zeros_acc)
    s2c = _combine_s(s2)

    pas3 = jnp.pad(a_src3.astype(_f32), ((0, HEADS - 1), (0, 0)))
    pad3 = jnp.pad(a_dst3.astype(_f32), ((0, HEADS - 1), (0, 0)))
    hcs3, as3, ad3, c3 = _k2(acc2, s2c, b2, g2, be2, W3, pas3, pad3,
                             erep, nout=1)
    ex3, s3 = _phase_b3(as3[:, 0], ad3[:, 0], c3.reshape(64),
                        src_i, dst_i)
    acc3p = _phase_c3(hcs3[0], ex3, src_i, dst_i, zeros_acc)
    s3c = s3.sum(axis=0)[:N]
    s3c8 = jnp.broadcast_to(s3c[:, None], (N, HEADS))
    return _k3(acc3p, s3c8, b3)
